# Initial kernel scaffold; baseline (speedup 1.0000x reference)
import jax
import jax.numpy as jnp
from jax.experimental import pallas as pl


def kernel(x, emb_weight):
    raise NotImplementedError("write your pallas kernel here")


# speedup vs baseline: 1.1875x; 1.1875x over previous
import functools

import jax
import jax.numpy as jnp
from jax import lax
from jax.experimental import pallas as pl
from jax.experimental.pallas import tpu as pltpu
from jax.experimental.pallas import tpu_sc as plsc

NUM_BUCKETS = 100000
EMBED_DIM = 128
BATCH = 16384

NUM_CORES = 2
NUM_SUBCORES = 16
NUM_WORKERS = NUM_CORES * NUM_SUBCORES
B_PER_W = BATCH // NUM_WORKERS
CHUNK = 128
NCHUNK = B_PER_W // CHUNK
LANES = 16


def _sc_embed_lookup(x_hbm, table_hbm, out_hbm, idx_v, hashed_v, rows_v, sem):
    wid = lax.axis_index("s") * NUM_CORES + lax.axis_index("c")
    base = wid * B_PER_W

    pltpu.sync_copy(x_hbm.at[wid], idx_v)

    for j in range(NCHUNK):
        for i in range(CHUNK // LANES):
            v = idx_v[j, pl.ds(i * LANES, LANES)]
            hashed_v[j, pl.ds(i * LANES, LANES)] = lax.rem(
                v, jnp.full((LANES,), NUM_BUCKETS, jnp.int32))

    cps = []
    for j in range(NCHUNK):
        cps.append(pltpu.async_copy(
            table_hbm.at[hashed_v.at[j]],
            rows_v.at[pl.ds(j * CHUNK, CHUNK)],
            sem))
    for cp in cps:
        cp.wait()

    pltpu.sync_copy(rows_v, out_hbm.at[pl.ds(base, B_PER_W)])


@jax.jit
def kernel(x, emb_weight):
    x3 = x.astype(jnp.int32).reshape(NUM_WORKERS, NCHUNK, CHUNK)
    mesh = plsc.VectorSubcoreMesh(
        core_axis_name="c", subcore_axis_name="s",
        num_cores=NUM_CORES, num_subcores=NUM_SUBCORES)
    f = functools.partial(
        pl.kernel,
        out_type=jax.ShapeDtypeStruct((BATCH, EMBED_DIM), jnp.float32),
        mesh=mesh,
        scratch_types=[
            pltpu.VMEM((NCHUNK, CHUNK), jnp.int32),
            pltpu.VMEM((NCHUNK, CHUNK), jnp.int32),
            pltpu.VMEM((B_PER_W, EMBED_DIM), jnp.float32),
            pltpu.SemaphoreType.DMA,
        ],
    )(_sc_embed_lookup)
    return f(x3, emb_weight)


# speedup vs baseline: 1.2575x; 1.0590x over previous
import functools

import jax
import jax.numpy as jnp
from jax import lax
from jax.experimental import pallas as pl
from jax.experimental.pallas import tpu as pltpu
from jax.experimental.pallas import tpu_sc as plsc

NUM_BUCKETS = 100000
EMBED_DIM = 128
BATCH = 16384

NUM_CORES = 2
NUM_SUBCORES = 16
NUM_WORKERS = NUM_CORES * NUM_SUBCORES
B_PER_W = BATCH // NUM_WORKERS
CHUNK = 128
NCHUNK = B_PER_W // CHUNK
LANES = 16


def _sc_embed_lookup(x_hbm, table_hbm, out_hbm, idx_v, hashed_v, rows_v, sem,
                     store_sem):
    wid = lax.axis_index("s") * NUM_CORES + lax.axis_index("c")
    base = wid * B_PER_W

    pltpu.sync_copy(x_hbm.at[wid], idx_v)

    gathers = []
    for j in range(NCHUNK):
        for i in range(CHUNK // LANES):
            v = idx_v[j, pl.ds(i * LANES, LANES)]
            hashed_v[j, pl.ds(i * LANES, LANES)] = lax.rem(
                v, jnp.full((LANES,), NUM_BUCKETS, jnp.int32))
        gathers.append(pltpu.async_copy(
            table_hbm.at[hashed_v.at[j]],
            rows_v.at[pl.ds(j * CHUNK, CHUNK)],
            sem))

    stores = []
    for j in range(NCHUNK):
        gathers[j].wait()
        stores.append(pltpu.async_copy(
            rows_v.at[pl.ds(j * CHUNK, CHUNK)],
            out_hbm.at[pl.ds(base + j * CHUNK, CHUNK)],
            store_sem))
    for cp in stores:
        cp.wait()


@jax.jit
def kernel(x, emb_weight):
    x3 = x.astype(jnp.int32).reshape(NUM_WORKERS, NCHUNK, CHUNK)
    mesh = plsc.VectorSubcoreMesh(
        core_axis_name="c", subcore_axis_name="s",
        num_cores=NUM_CORES, num_subcores=NUM_SUBCORES)
    f = functools.partial(
        pl.kernel,
        out_type=jax.ShapeDtypeStruct((BATCH, EMBED_DIM), jnp.float32),
        mesh=mesh,
        scratch_types=[
            pltpu.VMEM((NCHUNK, CHUNK), jnp.int32),
            pltpu.VMEM((NCHUNK, CHUNK), jnp.int32),
            pltpu.VMEM((B_PER_W, EMBED_DIM), jnp.float32),
            pltpu.SemaphoreType.DMA,
            pltpu.SemaphoreType.DMA,
        ],
    )(_sc_embed_lookup)
    return f(x3, emb_weight)


# speedup vs baseline: 1.5260x; 1.2135x over previous
import functools

import jax
import jax.numpy as jnp
from jax import lax
from jax.experimental import pallas as pl
from jax.experimental.pallas import tpu as pltpu
from jax.experimental.pallas import tpu_sc as plsc

NUM_BUCKETS = 100000
EMBED_DIM = 128
BATCH = 16384

NUM_CORES = 2
NUM_SUBCORES = 16
NUM_WORKERS = NUM_CORES * NUM_SUBCORES
B_PER_W = BATCH // NUM_WORKERS
CHUNK = 128
NCHUNK = B_PER_W // CHUNK
LANES = 16


def _sc_embed_lookup(x_hbm, table_hbm, out_hbm, idx_v, hashed_v, rows_v, sem,
                     store_sem):
    wid = lax.axis_index("s") * NUM_CORES + lax.axis_index("c")
    base = wid * B_PER_W

    pltpu.sync_copy(x_hbm.at[wid], idx_v)

    gathers = []
    for j in range(NCHUNK):
        for i in range(CHUNK // LANES):
            v = idx_v[j, pl.ds(i * LANES, LANES)]
            q = (v.astype(jnp.float32) * jnp.float32(1.0 / NUM_BUCKETS)
                 ).astype(jnp.int32)
            r = v - q * NUM_BUCKETS
            r = jnp.where(r < 0, r + NUM_BUCKETS, r)
            r = jnp.where(r >= NUM_BUCKETS, r - NUM_BUCKETS, r)
            hashed_v[j, pl.ds(i * LANES, LANES)] = r
        gathers.append(pltpu.async_copy(
            table_hbm.at[hashed_v.at[j]],
            rows_v.at[pl.ds(j * CHUNK, CHUNK)],
            sem))

    stores = []
    for j in range(NCHUNK):
        gathers[j].wait()
        stores.append(pltpu.async_copy(
            rows_v.at[pl.ds(j * CHUNK, CHUNK)],
            out_hbm.at[pl.ds(base + j * CHUNK, CHUNK)],
            store_sem))
    for cp in stores:
        cp.wait()


@jax.jit
def kernel(x, emb_weight):
    x3 = x.astype(jnp.int32).reshape(NUM_WORKERS, NCHUNK, CHUNK)
    mesh = plsc.VectorSubcoreMesh(
        core_axis_name="c", subcore_axis_name="s",
        num_cores=NUM_CORES, num_subcores=NUM_SUBCORES)
    f = functools.partial(
        pl.kernel,
        out_type=jax.ShapeDtypeStruct((BATCH, EMBED_DIM), jnp.float32),
        mesh=mesh,
        scratch_types=[
            pltpu.VMEM((NCHUNK, CHUNK), jnp.int32),
            pltpu.VMEM((NCHUNK, CHUNK), jnp.int32),
            pltpu.VMEM((B_PER_W, EMBED_DIM), jnp.float32),
            pltpu.SemaphoreType.DMA,
            pltpu.SemaphoreType.DMA,
        ],
    )(_sc_embed_lookup)
    return f(x3, emb_weight)
